# baseline (device time: 63253 ns/iter reference)
import jax
import jax.numpy as jnp
from jax import lax
from jax.experimental import pallas as pl
from jax.experimental.pallas import tpu as pltpu

N_DEV = 4
SQ = 256
SKV_SHARD = 4096
HQ = 8
DH = 128
DM = HQ * DH
CW = DM + DH
SCALE = 0.08838834764831843
HALF = SQ // 2
KG = SKV_SHARD // 4


def kernel(x, Wq, K_ext, V_ext, Wo):
    def body(x_ref, wq_ref, k_hbm, v_hbm, wo_buf, out_ref,
             comm, kperm, vperm, csem, ss, rs):
        my_pos = lax.axis_index("i")
        left = lax.rem(my_pos + N_DEV - 1, N_DEV)
        right = lax.rem(my_pos + 1, N_DEV)

        def kv_copies(j):
            cpk = pltpu.make_async_copy(
                k_hbm.at[0, :, j, :, :], kperm.at[j], csem.at[2 * j])
            cpv = pltpu.make_async_copy(
                v_hbm.at[0, :, j, :, :], vperm.at[j], csem.at[2 * j + 1])
            cpk.start()
            cpv.start()
            return cpk, cpv

        cur = kv_copies(0)

        barrier_sem = pltpu.get_barrier_semaphore()
        for nbr in (left, right):
            pl.semaphore_signal(
                barrier_sem, inc=1,
                device_id=(nbr,), device_id_type=pl.DeviceIdType.MESH,
            )
        pl.semaphore_wait(barrier_sem, 2)

        q = jnp.dot(x_ref[0], wq_ref[:, :],
                    preferred_element_type=jnp.float32) * SCALE

        def attend(j):
            rows = pl.ds(j * 64, 64)
            for h in range(HQ):
                qhj = q[j * 64:(j + 1) * 64, h * DH:(h + 1) * DH]
                khj = kperm[j, :, :, h * DH:(h + 1) * DH].reshape(KG, DH)
                vhj = vperm[j, :, :, h * DH:(h + 1) * DH].reshape(KG, DH)
                s = lax.dot_general(
                    qhj, khj, (((1,), (1,)), ((), ())),
                    preferred_element_type=jnp.float32)
                w = jnp.exp(s)
                l = jnp.sum(w, axis=1, keepdims=True)
                o = jnp.dot(w, vhj, preferred_element_type=jnp.float32)
                comm[0, rows, h * DH:(h + 1) * DH] = o.astype(jnp.bfloat16)
                comm[0, rows, DM + h:DM + h + 1] = l.astype(jnp.bfloat16)

        def mk(src_slot, dst_slot, row0, nrows, sem_i, dev):
            rows = pl.ds(row0, nrows)
            return pltpu.make_async_remote_copy(
                src_ref=comm.at[src_slot, rows, :],
                dst_ref=comm.at[dst_slot, rows, :],
                send_sem=ss.at[sem_i], recv_sem=rs.at[sem_i],
                device_id=(dev,), device_id_type=pl.DeviceIdType.MESH,
            )

        def combine(base):
            rows = pl.ds(base, HALF)
            tot = (comm[0, rows, :].astype(jnp.float32)
                   + comm[1, rows, :].astype(jnp.float32)
                   + comm[2, rows, :].astype(jnp.float32)
                   + comm[3, rows, :].astype(jnp.float32))
            ctx = jnp.concatenate(
                [tot[:, h * DH:(h + 1) * DH] / tot[:, DM + h:DM + h + 1]
                 for h in range(HQ)], axis=1)
            out_ref[0, rows, :] = jnp.dot(
                ctx, wo_buf[:, :], preferred_element_type=jnp.float32)

        cur[0].wait()
        cur[1].wait()
        cur = kv_copies(1)
        attend(0)
        cur[0].wait()
        cur[1].wait()
        cur = kv_copies(2)
        attend(1)

        r0a = mk(0, 1, 0, HALF, 0, right)
        l0a = mk(0, 2, 0, HALF, 1, left)
        r0a.start()
        l0a.start()

        cur[0].wait()
        cur[1].wait()
        cur = kv_copies(3)
        attend(2)

        r0a.wait_recv()
        r1a = mk(1, 3, 0, 64, 2, right)
        r1a.start()
        l0a.wait_recv()
        l1a = mk(2, 3, 64, 64, 3, left)
        l1a.start()

        cur[0].wait()
        cur[1].wait()
        attend(3)

        r0b = mk(0, 1, HALF, HALF, 4, right)
        l0b = mk(0, 2, HALF, HALF, 5, left)
        r0b.start()
        l0b.start()

        r1a.wait_recv()
        l1a.wait_recv()
        combine(0)

        r0b.wait_recv()
        r1b = mk(1, 3, HALF, 64, 6, right)
        r1b.start()
        l0b.wait_recv()
        l1b = mk(2, 3, HALF + 64, 64, 7, left)
        l1b.start()

        r1b.wait_recv()
        l1b.wait_recv()
        combine(HALF)

        for d in (r0a, l0a, r1a, l1a, r0b, l0b, r1b, l1b):
            d.wait_send()

    return pl.pallas_call(
        body,
        out_shape=jax.ShapeDtypeStruct((1, SQ, DM), jnp.float32),
        in_specs=[
            pl.BlockSpec(memory_space=pltpu.VMEM),
            pl.BlockSpec(memory_space=pltpu.VMEM),
            pl.BlockSpec(memory_space=pl.ANY),
            pl.BlockSpec(memory_space=pl.ANY),
            pl.BlockSpec(memory_space=pltpu.VMEM),
        ],
        out_specs=pl.BlockSpec(memory_space=pltpu.VMEM),
        scratch_shapes=[
            pltpu.VMEM((N_DEV, SQ, CW), jnp.bfloat16),
            pltpu.VMEM((4, 16, 64, DM), jnp.float32),
            pltpu.VMEM((4, 16, 64, DM), jnp.float32),
            pltpu.SemaphoreType.DMA((8,)),
            pltpu.SemaphoreType.DMA((8,)),
            pltpu.SemaphoreType.DMA((8,)),
        ],
        compiler_params=pltpu.CompilerParams(
            collective_id=0,
            vmem_limit_bytes=100 * 1024 * 1024,
        ),
    )(x, Wq,
      K_ext.reshape(1, 16, 4, 64, HQ * DH),
      V_ext.reshape(1, 16, 4, 64, HQ * DH),
      Wo)


# device time: 51265 ns/iter; 1.2338x vs baseline; 1.2338x over previous
import jax
import jax.numpy as jnp
from jax import lax
from jax.experimental import pallas as pl
from jax.experimental.pallas import tpu as pltpu

N_DEV = 4
SQ = 256
SKV_SHARD = 4096
HQ = 8
DH = 128
DM = HQ * DH
GW = 4 * DH
PW = GW + DH
CW = 2 * PW
SCALE = 0.08838834764831843
HALF = SQ // 2
KG = SKV_SHARD // 4


def kernel(x, Wq, K_ext, V_ext, Wo):
    def body(x_ref, wq_ref, k_ref, v_ref, wo_ref, out_ref,
             comm, ss, rs):
        my_pos = lax.axis_index("i")
        left = lax.rem(my_pos + N_DEV - 1, N_DEV)
        right = lax.rem(my_pos + 1, N_DEV)

        barrier_sem = pltpu.get_barrier_semaphore()
        for nbr in (left, right):
            pl.semaphore_signal(
                barrier_sem, inc=1,
                device_id=(nbr,), device_id_type=pl.DeviceIdType.MESH,
            )
        pl.semaphore_wait(barrier_sem, 2)

        q = jnp.dot(x_ref[0], wq_ref[:, :],
                    preferred_element_type=jnp.float32) * SCALE

        def attend(h):
            base = (h // 4) * PW
            hl = h % 4
            for j in range(4):
                qhj = q[j * 64:(j + 1) * 64, h * DH:(h + 1) * DH]
                khj = k_ref[0, :, j, :, h, :].reshape(KG, DH)
                vhj = v_ref[0, :, j, :, h, :].reshape(KG, DH)
                s = lax.dot_general(
                    qhj, khj, (((1,), (1,)), ((), ())),
                    preferred_element_type=jnp.float32)
                w = jnp.exp(s)
                l = jnp.sum(w, axis=1, keepdims=True)
                o = jnp.dot(w, vhj, preferred_element_type=jnp.float32)
                rows = pl.ds(j * 64, 64)
                comm[0, rows, base + hl * DH:base + (hl + 1) * DH] = (
                    o.astype(jnp.bfloat16))
                comm[0, rows, base + GW + hl:base + GW + hl + 1] = (
                    l.astype(jnp.bfloat16))

        def mk(src_slot, dst_slot, rows, lanes, sem_i, dev):
            return pltpu.make_async_remote_copy(
                src_ref=comm.at[src_slot, rows, lanes],
                dst_ref=comm.at[dst_slot, rows, lanes],
                send_sem=ss.at[sem_i], recv_sem=rs.at[sem_i],
                device_id=(dev,), device_id_type=pl.DeviceIdType.MESH,
            )

        full = pl.ds(0, SQ)
        top = pl.ds(0, HALF)
        bot = pl.ds(HALF, HALF)
        la = pl.ds(0, PW)
        lb = pl.ds(PW, PW)

        for h in range(4):
            attend(h)

        r0a = mk(0, 1, full, la, 0, right)
        l0a = mk(0, 2, full, la, 1, left)
        r0a.start()
        l0a.start()

        attend(4)
        attend(5)

        r0a.wait_recv()
        r1a = mk(1, 3, top, la, 2, right)
        r1a.start()
        l0a.wait_recv()
        l1a = mk(2, 3, bot, la, 3, left)
        l1a.start()

        attend(6)
        attend(7)

        r0b = mk(0, 1, full, lb, 4, right)
        l0b = mk(0, 2, full, lb, 5, left)
        r0b.start()
        l0b.start()

        r1a.wait_recv()
        l1a.wait_recv()
        sum_a = (comm[0, :, 0:PW].astype(jnp.float32)
                 + comm[1, :, 0:PW].astype(jnp.float32)
                 + comm[2, :, 0:PW].astype(jnp.float32)
                 + comm[3, :, 0:PW].astype(jnp.float32))
        ctx_a = jnp.concatenate(
            [sum_a[:, h * DH:(h + 1) * DH] / sum_a[:, GW + h:GW + h + 1]
             for h in range(4)], axis=1)
        out_a = jnp.dot(ctx_a, wo_ref[0:GW, :],
                        preferred_element_type=jnp.float32)

        r0b.wait_recv()
        r1b = mk(1, 3, top, lb, 6, right)
        r1b.start()
        l0b.wait_recv()
        l1b = mk(2, 3, bot, lb, 7, left)
        l1b.start()

        part_b = (comm[0, :, PW:CW].astype(jnp.float32)
                  + comm[1, :, PW:CW].astype(jnp.float32)
                  + comm[2, :, PW:CW].astype(jnp.float32))

        r1b.wait_recv()
        l1b.wait_recv()
        sum_b = part_b + comm[3, :, PW:CW].astype(jnp.float32)
        ctx_b = jnp.concatenate(
            [sum_b[:, h * DH:(h + 1) * DH] / sum_b[:, GW + h:GW + h + 1]
             for h in range(4)], axis=1)
        out_ref[0] = out_a + jnp.dot(
            ctx_b, wo_ref[GW:DM, :], preferred_element_type=jnp.float32)

        for d in (r0a, l0a, r1a, l1a, r0b, l0b, r1b, l1b):
            d.wait_send()

    return pl.pallas_call(
        body,
        out_shape=jax.ShapeDtypeStruct((1, SQ, DM), jnp.float32),
        in_specs=[pl.BlockSpec(memory_space=pltpu.VMEM)] * 5,
        out_specs=pl.BlockSpec(memory_space=pltpu.VMEM),
        scratch_shapes=[
            pltpu.VMEM((N_DEV, SQ, CW), jnp.bfloat16),
            pltpu.SemaphoreType.DMA((8,)),
            pltpu.SemaphoreType.DMA((8,)),
        ],
        compiler_params=pltpu.CompilerParams(
            collective_id=0,
            vmem_limit_bytes=100 * 1024 * 1024,
        ),
    )(x, Wq,
      K_ext.reshape(1, 16, 4, 64, HQ, DH),
      V_ext.reshape(1, 16, 4, 64, HQ, DH),
      Wo)
